# Initial kernel scaffold; baseline (speedup 1.0000x reference)
import jax
import jax.numpy as jnp
from jax.experimental import pallas as pl


def kernel(x, edge_index, edge_weight, W1, W2):
    raise NotImplementedError("write your pallas kernel here")


# speedup vs baseline: 3.1629x; 3.1629x over previous
import functools

import jax
import jax.numpy as jnp
from jax import lax
from jax.experimental import pallas as pl
from jax.experimental.pallas import tpu as pltpu
from jax.experimental.pallas import tpu_sc as plsc

N_NODES = 10000
NP = 10240
N_EDGES = 320000
CHUNK = 128
N_SUBCORES = 16
N_CORES = 2
EP = 32 * CHUNK * 79
CH_PER_SUB_1 = EP // (N_SUBCORES * CHUNK)
CH_PER_SUB_2 = EP // (N_CORES * N_SUBCORES * CHUNK)
ROWS_PER_SUB = NP // N_SUBCORES


def _hamilton(W1):
    r, i, j, k = jnp.split(W1, 4, axis=1)
    r2 = jnp.concatenate([r, -i, -j, -k], axis=0)
    i2 = jnp.concatenate([i, r, -k, j], axis=0)
    j2 = jnp.concatenate([j, k, r, -i], axis=0)
    k2 = jnp.concatenate([k, -j, i, r], axis=0)
    return jnp.concatenate([r2, i2, j2, k2], axis=1)


def _mm_body(a_ref, b_ref, o_ref):
    o_ref[...] = jnp.dot(a_ref[...], b_ref[...],
                         preferred_element_type=jnp.float32,
                         precision=lax.Precision.HIGHEST)


def _matmul(a, b, block_rows=2000):
    m, k = a.shape
    _, n = b.shape
    return pl.pallas_call(
        _mm_body,
        grid=(m // block_rows,),
        in_specs=[
            pl.BlockSpec((block_rows, k), lambda i: (i, 0)),
            pl.BlockSpec((k, n), lambda i: (0, 0)),
        ],
        out_specs=pl.BlockSpec((block_rows, n), lambda i: (i, 0)),
        out_shape=jax.ShapeDtypeStruct((m, n), jnp.float32),
    )(a, b)


def _mm2_body(ha_ref, hb_ref, wa_ref, wb_ref, o_ref):
    ha = jnp.maximum(ha_ref[...], 0.0)
    hb = jnp.maximum(hb_ref[...], 0.0)
    o_ref[...] = (
        jnp.dot(ha, wa_ref[...], preferred_element_type=jnp.float32,
                precision=lax.Precision.HIGHEST)
        + jnp.dot(hb, wb_ref[...], preferred_element_type=jnp.float32,
                  precision=lax.Precision.HIGHEST)
    )


def _relu_matmul2(ha, hb, w2a, w2b, block_rows=2000):
    m, k = ha.shape
    _, n = w2a.shape
    return pl.pallas_call(
        _mm2_body,
        grid=(m // block_rows,),
        in_specs=[
            pl.BlockSpec((block_rows, k), lambda i: (i, 0)),
            pl.BlockSpec((block_rows, k), lambda i: (i, 0)),
            pl.BlockSpec((k, n), lambda i: (0, 0)),
            pl.BlockSpec((k, n), lambda i: (0, 0)),
        ],
        out_specs=pl.BlockSpec((block_rows, n), lambda i: (i, 0)),
        out_shape=jax.ShapeDtypeStruct((m, n), jnp.float32),
    )(ha, hb, w2a, w2b)


def _final_body(p0_ref, p1_ref, o_ref):
    o = p0_ref[...][:, :64] + p1_ref[...][:, :64]
    m = jnp.max(o, axis=1, keepdims=True)
    e = jnp.exp(o - m)
    s = jnp.sum(e, axis=1, keepdims=True)
    o_ref[...] = (o - m) - jnp.log(s)


def _add_log_softmax(p0, p1, block_rows=2000):
    n = p0.shape[1]
    return pl.pallas_call(
        _final_body,
        grid=(N_NODES // block_rows,),
        in_specs=[
            pl.BlockSpec((block_rows, n), lambda i: (i, 0)),
            pl.BlockSpec((block_rows, n), lambda i: (i, 0)),
        ],
        out_specs=pl.BlockSpec((block_rows, 64), lambda i: (i, 0)),
        out_shape=jax.ShapeDtypeStruct((N_NODES, 64), jnp.float32),
    )(p0, p1)


def _zero_spmem(acc, rows, s, width):
    zero16 = jnp.zeros((16,), jnp.float32)

    def zbody(r, carry):
        for k in range(width // 16):
            rows[r, pl.ds(k * 16, 16)] = zero16
        return carry

    lax.fori_loop(0, CHUNK, zbody, 0)
    for j in range(ROWS_PER_SUB // CHUNK):
        pltpu.sync_copy(rows,
                        acc.at[pl.ds(s * ROWS_PER_SUB + j * CHUNK, CHUNK)])


def _scale_rows(rows, wv, width):
    def gbody(g, carry):
        w16 = wv[pl.ds(g * 16, 16)]
        for lane in range(16):
            wb = lax.broadcast(w16[lane], (16,))
            for k in range(width // 16):
                sl = pl.ds(k * 16, 16)
                rows[g * 16 + lane, sl] = rows[g * 16 + lane, sl] * wb
        return carry

    lax.fori_loop(0, CHUNK // 16, gbody, 0)


def _make_spmm1():
    mesh = plsc.VectorSubcoreMesh(core_axis_name="c", subcore_axis_name="s")

    @functools.partial(
        pl.kernel,
        mesh=mesh,
        out_type=[
            jax.ShapeDtypeStruct((NP, 128), jnp.float32),
            jax.ShapeDtypeStruct((NP, 128), jnp.float32),
        ],
        scratch_types=[
            pltpu.VMEM((CHUNK,), jnp.int32),
            pltpu.VMEM((CHUNK,), jnp.int32),
            pltpu.VMEM((CHUNK,), jnp.float32),
            pltpu.VMEM((CHUNK, 128), jnp.float32),
            pltpu.VMEM_SHARED((NP, 128), jnp.float32),
            pltpu.SemaphoreType.DMA,
        ],
    )
    def spmm1(sup_a, sup_b, col_hbm, row_hbm, w_hbm, out_a, out_b,
              colv, rowv, wv, rows, acc, sem):
        c = lax.axis_index("c")
        s = lax.axis_index("s")

        _zero_spmem(acc, rows, s, 128)
        plsc.subcore_barrier()

        def chunk(ci, carry):
            base = (s * CH_PER_SUB_1 + ci) * CHUNK
            pltpu.sync_copy(col_hbm.at[pl.ds(base, CHUNK)], colv)
            pltpu.sync_copy(row_hbm.at[pl.ds(base, CHUNK)], rowv)
            pltpu.sync_copy(w_hbm.at[pl.ds(base, CHUNK)], wv)

            @pl.when(c == 0)
            def _():
                pltpu.async_copy(sup_a.at[colv], rows, sem).wait()

            @pl.when(c == 1)
            def _():
                pltpu.async_copy(sup_b.at[colv], rows, sem).wait()

            _scale_rows(rows, wv, 128)
            pltpu.sync_copy(rows, acc.at[rowv], add=True)
            return carry

        lax.fori_loop(0, CH_PER_SUB_1, chunk, 0)
        plsc.subcore_barrier()

        @pl.when(c == 0)
        def _():
            pltpu.sync_copy(acc.at[pl.ds(s * ROWS_PER_SUB, ROWS_PER_SUB)],
                            out_a.at[pl.ds(s * ROWS_PER_SUB, ROWS_PER_SUB)])

        @pl.when(c == 1)
        def _():
            pltpu.sync_copy(acc.at[pl.ds(s * ROWS_PER_SUB, ROWS_PER_SUB)],
                            out_b.at[pl.ds(s * ROWS_PER_SUB, ROWS_PER_SUB)])

    return spmm1


def _make_spmm2():
    mesh = plsc.VectorSubcoreMesh(core_axis_name="c", subcore_axis_name="s")

    @functools.partial(
        pl.kernel,
        mesh=mesh,
        out_type=[
            jax.ShapeDtypeStruct((NP, 128), jnp.float32),
            jax.ShapeDtypeStruct((NP, 128), jnp.float32),
        ],
        scratch_types=[
            pltpu.VMEM((CHUNK,), jnp.int32),
            pltpu.VMEM((CHUNK,), jnp.int32),
            pltpu.VMEM((CHUNK,), jnp.float32),
            pltpu.VMEM((CHUNK, 128), jnp.float32),
            pltpu.VMEM_SHARED((NP, 128), jnp.float32),
            pltpu.SemaphoreType.DMA,
        ],
    )
    def spmm2(sup, col_hbm, row_hbm, w_hbm, out_p0, out_p1,
              colv, rowv, wv, rows, acc, sem):
        c = lax.axis_index("c")
        s = lax.axis_index("s")

        _zero_spmem(acc, rows, s, 128)
        plsc.subcore_barrier()

        def chunk(ci, carry):
            wid = c * N_SUBCORES + s
            base = (wid * CH_PER_SUB_2 + ci) * CHUNK
            pltpu.sync_copy(col_hbm.at[pl.ds(base, CHUNK)], colv)
            pltpu.sync_copy(row_hbm.at[pl.ds(base, CHUNK)], rowv)
            pltpu.sync_copy(w_hbm.at[pl.ds(base, CHUNK)], wv)
            pltpu.async_copy(sup.at[colv], rows, sem).wait()
            _scale_rows(rows, wv, 64)
            pltpu.sync_copy(rows, acc.at[rowv], add=True)
            return carry

        lax.fori_loop(0, CH_PER_SUB_2, chunk, 0)
        plsc.subcore_barrier()

        @pl.when(c == 0)
        def _():
            pltpu.sync_copy(acc.at[pl.ds(s * ROWS_PER_SUB, ROWS_PER_SUB)],
                            out_p0.at[pl.ds(s * ROWS_PER_SUB, ROWS_PER_SUB)])

        @pl.when(c == 1)
        def _():
            pltpu.sync_copy(acc.at[pl.ds(s * ROWS_PER_SUB, ROWS_PER_SUB)],
                            out_p1.at[pl.ds(s * ROWS_PER_SUB, ROWS_PER_SUB)])

    return spmm2


def kernel(x, edge_index, edge_weight, W1, W2):
    ham = _hamilton(W1)

    pad = EP - N_EDGES
    row = jnp.concatenate(
        [edge_index[0].astype(jnp.int32), jnp.zeros((pad,), jnp.int32)])
    col = jnp.concatenate(
        [edge_index[1].astype(jnp.int32), jnp.zeros((pad,), jnp.int32)])
    w = jnp.concatenate([edge_weight, jnp.zeros((pad,), jnp.float32)])

    support_a = _matmul(x, ham[:, :128])
    support_b = _matmul(x, ham[:, 128:])

    h_a, h_b = _make_spmm1()(support_a, support_b, col, row, w)

    w2p = jnp.concatenate([W2, jnp.zeros((W2.shape[0], 64), jnp.float32)], axis=1)
    s2 = _relu_matmul2(h_a, h_b, w2p[:128], w2p[128:], block_rows=1024)

    p0, p1 = _make_spmm2()(s2, col, row, w)

    return _add_log_softmax(p0, p1)
